# baseline (speedup 1.0000x reference)
import functools

import jax
import jax.numpy as jnp
from jax.experimental import pallas as pl
from jax.experimental.pallas import tpu as pltpu

M, B, DIN, DOUT = 8, 1024, 1024, 1024
MG = 2
BH = B // 2


def _mm_kernel(x1_ref, x2_ref, w_ref, b_ref, o_ref):
    for j in range(MG):
        w = w_ref[j].astype(jnp.bfloat16)
        for h, x_ref in ((0, x1_ref), (1, x2_ref)):
            x = x_ref[j].astype(jnp.bfloat16)
            acc = jax.lax.dot_general(
                x, w, (((1,), (0,)), ((), ())),
                preferred_element_type=jnp.float32,
            )
            o_ref[j, pl.ds(h * BH, BH), :] = acc + b_ref[j]


@functools.partial(jax.jit)
def kernel(inputs, W, b):
    grid = (M // MG,)
    out = pl.pallas_call(
        _mm_kernel,
        grid=grid,
        in_specs=[
            pl.BlockSpec((MG, BH, DIN), lambda m: (m, 0, 0)),
            pl.BlockSpec((MG, BH, DIN), lambda m: (m, 1, 0)),
            pl.BlockSpec((MG, DIN, DOUT), lambda m: (m, 0, 0)),
            pl.BlockSpec((MG, 1, DOUT), lambda m: (m, 0, 0)),
        ],
        out_specs=pl.BlockSpec((MG, B, DOUT), lambda m: (m, 0, 0)),
        out_shape=jax.ShapeDtypeStruct((M, B, DOUT), jnp.float32),
        compiler_params=pltpu.CompilerParams(
            dimension_semantics=("arbitrary",),
        ),
    )(inputs, inputs, W, b.reshape(M, 1, DOUT))
    return out


# speedup vs baseline: 1.0263x; 1.0263x over previous
import functools

import jax
import jax.numpy as jnp
from jax.experimental import pallas as pl
from jax.experimental.pallas import tpu as pltpu

M, B, DIN, DOUT = 8, 1024, 1024, 1024
MG = 2
NSTEP = M // MG


def _mm_kernel(x_ref, w_ref, b_ref, o_hbm, acc_ref, sems):
    m = pl.program_id(0)
    par = jax.lax.rem(m, 2)

    for j in range(MG):
        @pl.when(m >= 2)
        def _():
            pltpu.make_async_copy(
                acc_ref.at[par, j], o_hbm.at[MG * (m - 2) + j], sems.at[par, j]
            ).wait()

        x = x_ref[j].astype(jnp.bfloat16)
        w = w_ref[j].astype(jnp.bfloat16)
        acc = jax.lax.dot_general(
            x, w, (((1,), (0,)), ((), ())),
            preferred_element_type=jnp.float32,
        )
        acc_ref[par, j] = acc + b_ref[j]
        pltpu.make_async_copy(
            acc_ref.at[par, j], o_hbm.at[MG * m + j], sems.at[par, j]
        ).start()

    @pl.when(m == NSTEP - 1)
    def _():
        for j in range(MG):
            pltpu.make_async_copy(
                acc_ref.at[1 - par, j], o_hbm.at[MG * (m - 1) + j],
                sems.at[1 - par, j],
            ).wait()
            pltpu.make_async_copy(
                acc_ref.at[par, j], o_hbm.at[MG * m + j], sems.at[par, j]
            ).wait()


@functools.partial(jax.jit)
def kernel(inputs, W, b):
    out = pl.pallas_call(
        _mm_kernel,
        grid=(NSTEP,),
        in_specs=[
            pl.BlockSpec((MG, B, DIN), lambda m: (m, 0, 0)),
            pl.BlockSpec((MG, DIN, DOUT), lambda m: (m, 0, 0)),
            pl.BlockSpec((MG, 1, DOUT), lambda m: (m, 0, 0)),
        ],
        out_specs=pl.BlockSpec(memory_space=pltpu.MemorySpace.HBM),
        out_shape=jax.ShapeDtypeStruct((M, B, DOUT), jnp.float32),
        scratch_shapes=[
            pltpu.VMEM((2, MG, B, DOUT), jnp.float32),
            pltpu.SemaphoreType.DMA((2, MG)),
        ],
        compiler_params=pltpu.CompilerParams(
            dimension_semantics=("arbitrary",),
        ),
    )(inputs, W, b.reshape(M, 1, DOUT))
    return out
